# Initial kernel scaffold; baseline (speedup 1.0000x reference)
import jax
import jax.numpy as jnp
from jax.experimental import pallas as pl


def kernel(h_query, memory_embeds, pred_values):
    raise NotImplementedError("write your pallas kernel here")


# speedup vs baseline: 1.7489x; 1.7489x over previous
import functools

import jax
import jax.numpy as jnp
from jax import lax
from jax.experimental import pallas as pl
from jax.experimental.pallas import tpu as pltpu
from jax.experimental.pallas import tpu_sc as plsc

_Q = 4096
_K = 100000
_D = 256
_TOPK = 3

_BK = 2048
_NKB = 49
_KPAD = _NKB * _BK
_BQ = 256
_NQB = _Q // _BQ

_BIG = 1e30
_IBIG = 2**30


def _ce(a, b):
    keep = a[0] <= b[0]
    lo = (jnp.where(keep, a[0], b[0]), jnp.where(keep, a[1], b[1]))
    hi = (jnp.where(keep, b[0], a[0]), jnp.where(keep, b[1], a[1]))
    return lo, hi


def _lo(a, b):
    keep = a[0] <= b[0]
    return (jnp.where(keep, a[0], b[0]), jnp.where(keep, a[1], b[1]))


def _topk_body(q_ref, qsq_ref, m2_ref, msq_ref, aux_ref,
               v0_ref, v1_ref, v2_ref, i0_ref, i1_ref, i2_ref):
    kj = pl.program_id(0)

    @pl.when(kj == 0)
    def _init():
        for vr in (v0_ref, v1_ref, v2_ref):
            vr[...] = jnp.full((_Q, 1), _BIG, jnp.float32)
        for ir in (i0_ref, i1_ref, i2_ref):
            ir[...] = jnp.full((_Q, 1), _IBIG, jnp.int32)

    m2_blk = m2_ref[...]
    msq = msq_ref[...]
    aux = aux_ref[...]
    kbase = kj * _BK

    for i in range(_NQB):
        rows = pl.ds(i * _BQ, _BQ)
        q = q_ref[rows, :]
        qsq = qsq_ref[rows, :]
        dot2 = lax.dot_general(q, m2_blk, (((1,), (1,)), ((), ())),
                               preferred_element_type=jnp.float32)
        s0 = (qsq + msq) + dot2
        s = s0

        cands = []
        any_tie = jnp.int32(0)
        for t in range(_TOPK):
            bmin = jnp.min(s, axis=1, keepdims=True)
            eqb = s == bmin
            eqf = jnp.where(eqb, 1.0, 0.0).astype(jnp.float32)
            sc = lax.dot_general(eqf, aux, (((1,), (0,)), ((), ())),
                                 preferred_element_type=jnp.float32)
            cnt = sc[:, 2:3]
            bidx = (sc[:, 0:1].astype(jnp.int32) * 128
                    + sc[:, 1:2].astype(jnp.int32) + kbase)
            cands.append((bmin, bidx))
            any_tie = jnp.maximum(any_tie,
                                  jnp.max(jnp.where(cnt > 1.5, 1, 0)))
            if t < _TOPK - 1:
                s = jnp.where(eqb, _BIG, s)

        def merge(b0, b1, b2):
            a0 = (v0_ref[rows, :], i0_ref[rows, :])
            a1 = (v1_ref[rows, :], i1_ref[rows, :])
            a2 = (v2_ref[rows, :], i2_ref[rows, :])
            a2 = _lo(a2, b0)
            a1, a2 = _ce(a1, a2)
            a0, a1 = _ce(a0, a1)
            a2 = _lo(a2, b1)
            a1, a2 = _ce(a1, a2)
            a2 = _lo(a2, b2)
            v0_ref[rows, :], i0_ref[rows, :] = a0
            v1_ref[rows, :], i1_ref[rows, :] = a1
            v2_ref[rows, :], i2_ref[rows, :] = a2

        @pl.when(any_tie == 0)
        def _fast():
            merge(*cands)

        @pl.when(any_tie != 0)
        def _exact():
            col = lax.broadcasted_iota(jnp.int32, (_BQ, _BK), 1) + kbase
            se = s0
            cs = []
            for t in range(_TOPK):
                bmin = jnp.min(se, axis=1, keepdims=True)
                bidx = jnp.min(jnp.where(se == bmin, col, _IBIG), axis=1,
                               keepdims=True)
                cs.append((bmin, bidx))
                if t < _TOPK - 1:
                    se = jnp.where(col == bidx, _BIG, se)
            merge(*cs)


def _tc_topk(h_query, qsq, m2_pad, msq_pad, aux):
    out_specs = [pl.BlockSpec((_Q, 1), lambda kj: (0, 0)) for _ in range(6)]
    out_shape = ([jax.ShapeDtypeStruct((_Q, 1), jnp.float32)] * 3
                 + [jax.ShapeDtypeStruct((_Q, 1), jnp.int32)] * 3)
    return pl.pallas_call(
        _topk_body,
        grid=(_NKB,),
        in_specs=[
            pl.BlockSpec((_Q, _D), lambda kj: (0, 0)),
            pl.BlockSpec((_Q, 1), lambda kj: (0, 0)),
            pl.BlockSpec((_BK, _D), lambda kj: (kj, 0)),
            pl.BlockSpec((1, _BK), lambda kj: (0, kj)),
            pl.BlockSpec((_BK, 128), lambda kj: (0, 0)),
        ],
        out_specs=out_specs,
        out_shape=out_shape,
    )(h_query, qsq, m2_pad, msq_pad, aux)


_NW = 32
_ROWS_PER_W = 3


@functools.lru_cache(maxsize=1)
def _sc_gather_sum_fn():
    mesh = plsc.VectorSubcoreMesh(core_axis_name="c", subcore_axis_name="s")

    @functools.partial(
        pl.kernel,
        mesh=mesh,
        out_type=jax.ShapeDtypeStruct((_NW * 16,), jnp.float32),
        scratch_types=[
            pltpu.VMEM((_ROWS_PER_W, 128), jnp.int32),
            pltpu.VMEM((_ROWS_PER_W, 128), jnp.float32),
            pltpu.VMEM((16,), jnp.float32),
            pltpu.SemaphoreType.DMA,
        ],
    )
    def _sc_gather_sum(idx_hbm, pred_hbm, out_hbm, idx_v, vals_v, acc_v, sem):
        c = lax.axis_index("c")
        s = lax.axis_index("s")
        wid = s * 2 + c
        base = wid * (_ROWS_PER_W * 128)
        for j in range(_ROWS_PER_W):
            pltpu.sync_copy(idx_hbm.at[pl.ds(base + j * 128, 128)],
                            idx_v.at[j])
        copies = [
            pltpu.async_copy(pred_hbm.at[idx_v.at[j]], vals_v.at[j], sem)
            for j in range(_ROWS_PER_W)
        ]
        for cp in copies:
            cp.wait()
        acc = jnp.zeros((16,), jnp.float32)
        for j in range(_ROWS_PER_W):
            for g in range(8):
                acc = acc + vals_v[j, pl.ds(g * 16, 16)]
        acc_v[...] = acc
        pltpu.sync_copy(acc_v, out_hbm.at[pl.ds(wid * 16, 16)])

    return _sc_gather_sum


def kernel(h_query, memory_embeds, pred_values):
    qsq = jnp.sum(h_query * h_query, axis=1, keepdims=True)
    msq = jnp.sum(memory_embeds * memory_embeds, axis=1)
    hq_bf = h_query.astype(jnp.bfloat16)
    m2_pad = jnp.pad((memory_embeds * -2.0).astype(jnp.bfloat16),
                     ((0, _KPAD - _K), (0, 0)))
    msq_pad = jnp.pad(msq, (0, _KPAD - _K),
                      constant_values=_BIG).reshape(1, _KPAD)
    ids = jnp.arange(_BK, dtype=jnp.int32)
    aux = jnp.concatenate(
        [(ids // 128).astype(jnp.float32).reshape(_BK, 1),
         (ids % 128).astype(jnp.float32).reshape(_BK, 1),
         jnp.ones((_BK, 1), jnp.float32),
         jnp.zeros((_BK, 125), jnp.float32)], axis=1)

    outs = _tc_topk(hq_bf, qsq, m2_pad, msq_pad, aux)
    i0, i1, i2 = outs[3], outs[4], outs[5]
    idx = jnp.concatenate([i0, i1, i2], axis=1).reshape(-1)

    parts = _sc_gather_sum_fn()(idx, pred_values)
    return jnp.sum(parts) / jnp.float32(_Q * _TOPK)


# speedup vs baseline: 3.9681x; 2.2690x over previous
import functools

import jax
import jax.numpy as jnp
from jax import lax
from jax.experimental import pallas as pl
from jax.experimental.pallas import tpu as pltpu
from jax.experimental.pallas import tpu_sc as plsc

_Q = 4096
_K = 100000
_D = 256
_TOPK = 3

_BK = 2048
_NKB = 49
_KPAD = _NKB * _BK
_BQ = 256
_NQB = _Q // _BQ

_BIG = 1e30
_FBIG = float(2**30)


def _ce(a, b):
    keep = a[0] <= b[0]
    lo = (jnp.where(keep, a[0], b[0]), jnp.where(keep, a[1], b[1]))
    hi = (jnp.where(keep, b[0], a[0]), jnp.where(keep, b[1], a[1]))
    return lo, hi


def _lo(a, b):
    keep = a[0] <= b[0]
    return (jnp.where(keep, a[0], b[0]), jnp.where(keep, a[1], b[1]))


def _topk_body(q_ref, qsq_ref, m2_ref, msq_ref,
               v0_ref, v1_ref, v2_ref, i0_ref, i1_ref, i2_ref):
    kj = pl.program_id(0)

    @pl.when(kj == 0)
    def _init():
        for vr in (v0_ref, v1_ref, v2_ref):
            vr[...] = jnp.full((_Q, 1), _BIG, jnp.float32)
        for ir in (i0_ref, i1_ref, i2_ref):
            ir[...] = jnp.full((_Q, 1), _FBIG, jnp.float32)

    m2_blk = m2_ref[...]
    msq = msq_ref[...]
    kbase = kj * _BK
    colf = (lax.broadcasted_iota(jnp.int32, (_BQ, _BK), 1).astype(jnp.float32)
            + jnp.float32(kbase))

    for i in range(_NQB):
        rows = pl.ds(i * _BQ, _BQ)
        q = q_ref[rows, :]
        qsq = qsq_ref[rows, :]
        dot2 = lax.dot_general(q, m2_blk, (((1,), (1,)), ((), ())),
                               preferred_element_type=jnp.float32)
        s = (qsq + msq) + dot2

        cands = []
        for t in range(_TOPK):
            bmin = jnp.min(s, axis=1, keepdims=True)
            bidx = jnp.min(jnp.where(s == bmin, colf, _FBIG), axis=1,
                           keepdims=True)
            cands.append((bmin, bidx))
            if t < _TOPK - 1:
                s = jnp.where(colf == bidx, _BIG, s)

        a0 = (v0_ref[rows, :], i0_ref[rows, :])
        a1 = (v1_ref[rows, :], i1_ref[rows, :])
        a2 = (v2_ref[rows, :], i2_ref[rows, :])
        b0, b1, b2 = cands

        a2 = _lo(a2, b0)
        a1, a2 = _ce(a1, a2)
        a0, a1 = _ce(a0, a1)
        a2 = _lo(a2, b1)
        a1, a2 = _ce(a1, a2)
        a2 = _lo(a2, b2)

        v0_ref[rows, :], i0_ref[rows, :] = a0
        v1_ref[rows, :], i1_ref[rows, :] = a1
        v2_ref[rows, :], i2_ref[rows, :] = a2


def _tc_topk(h_query, qsq, m2_pad, msq_pad):
    out_specs = [pl.BlockSpec((_Q, 1), lambda kj: (0, 0)) for _ in range(6)]
    out_shape = [jax.ShapeDtypeStruct((_Q, 1), jnp.float32)] * 6
    return pl.pallas_call(
        _topk_body,
        grid=(_NKB,),
        in_specs=[
            pl.BlockSpec((_Q, _D), lambda kj: (0, 0)),
            pl.BlockSpec((_Q, 1), lambda kj: (0, 0)),
            pl.BlockSpec((_BK, _D), lambda kj: (kj, 0)),
            pl.BlockSpec((1, _BK), lambda kj: (0, kj)),
        ],
        out_specs=out_specs,
        out_shape=out_shape,
    )(h_query, qsq, m2_pad, msq_pad)


_NW = 32
_ROWS_PER_W = 3


@functools.lru_cache(maxsize=1)
def _sc_gather_sum_fn():
    mesh = plsc.VectorSubcoreMesh(core_axis_name="c", subcore_axis_name="s")

    @functools.partial(
        pl.kernel,
        mesh=mesh,
        out_type=jax.ShapeDtypeStruct((_NW * 16,), jnp.float32),
        scratch_types=[
            pltpu.VMEM((_ROWS_PER_W, 128), jnp.int32),
            pltpu.VMEM((_ROWS_PER_W, 128), jnp.float32),
            pltpu.VMEM((16,), jnp.float32),
            pltpu.SemaphoreType.DMA,
        ],
    )
    def _sc_gather_sum(idx_hbm, pred_hbm, out_hbm, idx_v, vals_v, acc_v, sem):
        c = lax.axis_index("c")
        s = lax.axis_index("s")
        wid = s * 2 + c
        base = wid * (_ROWS_PER_W * 128)
        for j in range(_ROWS_PER_W):
            pltpu.sync_copy(idx_hbm.at[pl.ds(base + j * 128, 128)],
                            idx_v.at[j])
        copies = [
            pltpu.async_copy(pred_hbm.at[idx_v.at[j]], vals_v.at[j], sem)
            for j in range(_ROWS_PER_W)
        ]
        for cp in copies:
            cp.wait()
        acc = jnp.zeros((16,), jnp.float32)
        for j in range(_ROWS_PER_W):
            for g in range(8):
                acc = acc + vals_v[j, pl.ds(g * 16, 16)]
        acc_v[...] = acc
        pltpu.sync_copy(acc_v, out_hbm.at[pl.ds(wid * 16, 16)])

    return _sc_gather_sum


def kernel(h_query, memory_embeds, pred_values):
    qsq = jnp.sum(h_query * h_query, axis=1, keepdims=True)
    msq = jnp.sum(memory_embeds * memory_embeds, axis=1)
    hq_bf = h_query.astype(jnp.bfloat16)
    m2_pad = jnp.pad((memory_embeds * -2.0).astype(jnp.bfloat16),
                     ((0, _KPAD - _K), (0, 0)))
    msq_pad = jnp.pad(msq, (0, _KPAD - _K),
                      constant_values=_BIG).reshape(1, _KPAD)
    outs = _tc_topk(hq_bf, qsq, m2_pad, msq_pad)
    i0, i1, i2 = outs[3], outs[4], outs[5]
    idx = jnp.concatenate([i0, i1, i2], axis=1).astype(jnp.int32).reshape(-1)

    parts = _sc_gather_sum_fn()(idx, pred_values)
    return jnp.sum(parts) / jnp.float32(_Q * _TOPK)
